# baseline (speedup 1.0000x reference)
import functools

import jax
import jax.numpy as jnp
from jax import lax
from jax.experimental import pallas as pl
from jax.experimental.pallas import tpu as pltpu
from jax.experimental.pallas import tpu_sc as plsc

B = 4
N = 65536
M = N * 3
GRP = 8
CH = M // GRP
PCH = N // GRP
L = 16
RCH = 1024
NCK = PCH // RCH
CEL = RCH * 3
UN = 8
EPS = 1e-07
_MASK31 = 0x7FFFFFFF


def _sc_point_loss(pred2, target2, weight_f):
    mesh = plsc.VectorSubcoreMesh(core_axis_name="c", subcore_axis_name="s")

    @functools.partial(
        pl.kernel,
        mesh=mesh,
        out_type=jax.ShapeDtypeStruct((B * L,), jnp.float32),
        compiler_params=pltpu.CompilerParams(
            needs_layout_passes=False, use_tc_tiling_on_sc=False),
        scratch_types=[
            pltpu.VMEM((RCH, 3), jnp.float32),
            pltpu.VMEM((RCH, 3), jnp.float32),
            pltpu.VMEM((PCH,), jnp.float32),
            pltpu.VMEM((CH,), jnp.int32),
            pltpu.VMEM((CH,), jnp.float32),
            pltpu.VMEM((L,), jnp.float32),
            pltpu.VMEM((GRP * L,), jnp.float32),
            pltpu.VMEM((L,), jnp.float32),
            pltpu.VMEM_SHARED((2, GRP * L), jnp.float32),
        ],
    )
    def k(pred_hbm, target_hbm, weight_hbm, out_hbm,
          p_buf, t_buf, w_v, key_v, wx_v, stage_v, grp_v, out_v, shared):
        cid = lax.axis_index("c")
        sid = lax.axis_index("s")
        g = sid // GRP
        lid = sid % GRP
        b = cid * 2 + g
        lane = lax.iota(jnp.int32, L)

        r0 = b * N + lid * PCH
        pltpu.sync_copy(weight_hbm.at[pl.ds(r0, PCH)], w_v)

        zero = jnp.zeros((L,), jnp.float32)
        eps = jnp.float32(EPS)

        def global_sum(vec):
            stage_v[...] = vec
            plsc.subcore_barrier()
            pltpu.sync_copy(stage_v, shared.at[g, pl.ds(lid * L, L)])
            plsc.subcore_barrier()
            pltpu.sync_copy(shared.at[g], grp_v)

            def rd(j, acc):
                return acc + grp_v[pl.ds(j * L, L)]

            return jnp.sum(lax.fori_loop(0, GRP, rd, zero))

        def pass_a_chunk(c, acc):
            pltpu.sync_copy(pred_hbm.at[pl.ds(r0 + c * RCH, RCH)], p_buf)
            pltpu.sync_copy(target_hbm.at[pl.ds(r0 + c * RCH, RCH)], t_buf)

            def body(i, acc):
                for u in range(UN):
                    el = (i * UN + u) * L + lane
                    row = el // 3
                    col = el - row * 3
                    p = plsc.load_gather(p_buf, [row, col])
                    t = plsc.load_gather(t_buf, [row, col])
                    w = plsc.load_gather(w_v, [c * RCH + row])
                    sgn = jnp.where(
                        p >= 0.0, jnp.float32(1.0), jnp.float32(-1.0))
                    xa = jnp.abs(p)
                    ya = t * sgn
                    ratio = ya / jnp.maximum(xa, eps)
                    bits = plsc.bitcast(ratio, jnp.int32)
                    key = jnp.where(
                        bits >= 0, bits, bits ^ jnp.int32(_MASK31))
                    sl = pl.ds(c * CEL + (i * UN + u) * L, L)
                    key_v[sl] = key
                    wx_v[sl] = xa * w
                    acc = acc + xa * w
                return acc

            return lax.fori_loop(0, CEL // (UN * L), body, acc)

        tvec = lax.fori_loop(0, NCK, pass_a_chunk, zero)
        t_half = global_sum(tvec) * jnp.float32(0.5)

        def count_lt(q):
            qv = jnp.full((L,), q, jnp.int32)

            def body(i, acc):
                for u in range(UN):
                    sl = pl.ds((i * UN + u) * L, L)
                    kk = key_v[sl]
                    vv = wx_v[sl]
                    acc = acc + jnp.where(kk < qv, vv, jnp.float32(0.0))
                return acc

            return lax.fori_loop(0, CH // (UN * L), body, zero)

        c0 = global_sum(count_lt(jnp.int32(0)))
        p_key = jnp.where(c0 < t_half, jnp.int32(0), jnp.int32(-2147483648))

        def round_body(r, p_key):
            q = p_key + (jnp.int32(1) << (30 - r))
            c = global_sum(count_lt(q))
            return jnp.where(c < t_half, q, p_key)

        p_key = lax.fori_loop(0, 31, round_body, p_key)

        pbits = jnp.where(p_key >= 0, p_key, p_key ^ jnp.int32(_MASK31))
        a_vec = plsc.bitcast(jnp.full((L,), pbits, jnp.int32), jnp.float32)

        def pass_c_chunk(c, acc):
            pltpu.sync_copy(pred_hbm.at[pl.ds(r0 + c * RCH, RCH)], p_buf)
            pltpu.sync_copy(target_hbm.at[pl.ds(r0 + c * RCH, RCH)], t_buf)

            def body(i, acc):
                for u in range(UN):
                    el = (i * UN + u) * L + lane
                    row = el // 3
                    col = el - row * 3
                    p = plsc.load_gather(p_buf, [row, col])
                    t = plsc.load_gather(t_buf, [row, col])
                    w = plsc.load_gather(w_v, [c * RCH + row])
                    acc = acc + w * jnp.abs(a_vec * p - t)
                return acc

            return lax.fori_loop(0, CEL // (UN * L), body, acc)

        num_vec = lax.fori_loop(0, NCK, pass_c_chunk, zero)

        def pass_w(i, acc):
            return acc + w_v[pl.ds(i * L, L)]

        den_vec = lax.fori_loop(0, PCH // L, pass_w, zero)

        num = global_sum(num_vec)
        den = global_sum(den_vec)

        @pl.when(lid == 0)
        def _():
            out_v[...] = jnp.where(
                lane == 0, num, jnp.where(lane == 1, den, jnp.float32(0.0)))
            pltpu.sync_copy(out_v, out_hbm.at[pl.ds(b * L, L)])

    return k(pred2, target2, weight_f)


def kernel(pred, target, weight):
    pred2 = pred.reshape(B * N, 3)
    target2 = target.reshape(B * N, 3)
    weight_f = weight.reshape(B * N)
    out = _sc_point_loss(pred2, target2, weight_f).reshape(B, L)
    per_batch = out[:, 0]
    denom = 3.0 * jnp.maximum(out[:, 1], EPS)
    return jnp.mean(per_batch / denom)


# speedup vs baseline: 1.0584x; 1.0584x over previous
import functools

import jax
import jax.numpy as jnp
from jax import lax
from jax.experimental import pallas as pl
from jax.experimental.pallas import tpu as pltpu
from jax.experimental.pallas import tpu_sc as plsc

B = 4
N = 65536
M = N * 3
GRP = 8
CH = M // GRP
PCH = N // GRP
L = 16
RCH = 128
NCK = PCH // RCH
CEL = RCH * 3
UN = 8
EPS = 1e-07
_MASK31 = 0x7FFFFFFF


def _sc_point_loss(pred, target, weight):
    mesh = plsc.VectorSubcoreMesh(core_axis_name="c", subcore_axis_name="s")

    @functools.partial(
        pl.kernel,
        mesh=mesh,
        out_type=jax.ShapeDtypeStruct((B * L,), jnp.float32),
        compiler_params=pltpu.CompilerParams(needs_layout_passes=False),
        scratch_types=[
            pltpu.VMEM((RCH, 3), jnp.float32),
            pltpu.VMEM((RCH, 3), jnp.float32),
            pltpu.VMEM((PCH,), jnp.float32),
            pltpu.VMEM((CH,), jnp.int32),
            pltpu.VMEM((CH,), jnp.float32),
            pltpu.VMEM((L,), jnp.float32),
            pltpu.VMEM((GRP * L,), jnp.float32),
            pltpu.VMEM((L,), jnp.float32),
            pltpu.VMEM_SHARED((2, GRP * L), jnp.float32),
        ],
    )
    def k(pred_hbm, target_hbm, weight_hbm, out_hbm,
          p_buf, t_buf, w_v, key_v, wx_v, stage_v, grp_v, out_v, shared):
        cid = lax.axis_index("c")
        sid = lax.axis_index("s")
        g = sid // GRP
        lid = sid % GRP
        b = cid * 2 + g
        lane = lax.iota(jnp.int32, L)

        n0 = lid * PCH
        pltpu.sync_copy(weight_hbm.at[b, pl.ds(n0, PCH)], w_v)

        zero = jnp.zeros((L,), jnp.float32)
        eps = jnp.float32(EPS)

        def global_sum(vec):
            stage_v[...] = vec
            plsc.subcore_barrier()
            pltpu.sync_copy(stage_v, shared.at[g, pl.ds(lid * L, L)])
            plsc.subcore_barrier()
            pltpu.sync_copy(shared.at[g], grp_v)

            def rd(j, acc):
                return acc + grp_v[pl.ds(j * L, L)]

            return jnp.sum(lax.fori_loop(0, GRP, rd, zero))

        def pass_a_chunk(c, acc):
            pltpu.sync_copy(pred_hbm.at[b, pl.ds(n0 + c * RCH, RCH)], p_buf)
            pltpu.sync_copy(target_hbm.at[b, pl.ds(n0 + c * RCH, RCH)], t_buf)

            def body(i, acc):
                for u in range(UN):
                    el = (i * UN + u) * L + lane
                    row = el // 3
                    col = el - row * 3
                    p = plsc.load_gather(p_buf, [row, col])
                    t = plsc.load_gather(t_buf, [row, col])
                    w = plsc.load_gather(w_v, [c * RCH + row])
                    sgn = jnp.where(
                        p >= 0.0, jnp.float32(1.0), jnp.float32(-1.0))
                    xa = jnp.abs(p)
                    ya = t * sgn
                    ratio = ya / jnp.maximum(xa, eps)
                    bits = plsc.bitcast(ratio, jnp.int32)
                    key = jnp.where(
                        bits >= 0, bits, bits ^ jnp.int32(_MASK31))
                    sl = pl.ds(c * CEL + (i * UN + u) * L, L)
                    key_v[sl] = key
                    wx_v[sl] = xa * w
                    acc = acc + xa * w
                return acc

            return lax.fori_loop(0, CEL // (UN * L), body, acc)

        tvec = lax.fori_loop(0, NCK, pass_a_chunk, zero)
        t_half = global_sum(tvec) * jnp.float32(0.5)

        def count_lt(q):
            qv = jnp.full((L,), q, jnp.int32)

            def body(i, acc):
                for u in range(UN):
                    sl = pl.ds((i * UN + u) * L, L)
                    kk = key_v[sl]
                    vv = wx_v[sl]
                    acc = acc + jnp.where(kk < qv, vv, jnp.float32(0.0))
                return acc

            return lax.fori_loop(0, CH // (UN * L), body, zero)

        c0 = global_sum(count_lt(jnp.int32(0)))
        p_key = jnp.where(c0 < t_half, jnp.int32(0), jnp.int32(-2147483648))

        def round_body(r, p_key):
            q = p_key + (jnp.int32(1) << (30 - r))
            c = global_sum(count_lt(q))
            return jnp.where(c < t_half, q, p_key)

        p_key = lax.fori_loop(0, 31, round_body, p_key)

        pbits = jnp.where(p_key >= 0, p_key, p_key ^ jnp.int32(_MASK31))
        a_vec = plsc.bitcast(jnp.full((L,), pbits, jnp.int32), jnp.float32)

        def pass_c_chunk(c, acc):
            pltpu.sync_copy(pred_hbm.at[b, pl.ds(n0 + c * RCH, RCH)], p_buf)
            pltpu.sync_copy(target_hbm.at[b, pl.ds(n0 + c * RCH, RCH)], t_buf)

            def body(i, acc):
                for u in range(UN):
                    el = (i * UN + u) * L + lane
                    row = el // 3
                    col = el - row * 3
                    p = plsc.load_gather(p_buf, [row, col])
                    t = plsc.load_gather(t_buf, [row, col])
                    w = plsc.load_gather(w_v, [c * RCH + row])
                    acc = acc + w * jnp.abs(a_vec * p - t)
                return acc

            return lax.fori_loop(0, CEL // (UN * L), body, acc)

        num_vec = lax.fori_loop(0, NCK, pass_c_chunk, zero)

        def pass_w(i, acc):
            return acc + w_v[pl.ds(i * L, L)]

        den_vec = lax.fori_loop(0, PCH // L, pass_w, zero)

        num = global_sum(num_vec)
        den = global_sum(den_vec)

        @pl.when(lid == 0)
        def _():
            out_v[...] = jnp.where(
                lane == 0, num, jnp.where(lane == 1, den, jnp.float32(0.0)))
            pltpu.sync_copy(out_v, out_hbm.at[pl.ds(b * L, L)])

    return k(pred, target, weight)


def kernel(pred, target, weight):
    out = _sc_point_loss(pred, target, weight).reshape(B, L)
    per_batch = out[:, 0]
    denom = 3.0 * jnp.maximum(out[:, 1], EPS)
    return jnp.mean(per_batch / denom)
